# baseline (speedup 1.0000x reference)
import jax
import jax.numpy as jnp
from jax import lax
from jax.experimental import pallas as pl
from jax.experimental.pallas import tpu as pltpu
from jax.experimental.pallas import tpu_sc as plsc

NC, NS, L = 2, 16, 16
NW = NC * NS
BATCH = 16384
DIM = 16
NROW = 1000000
BPW = BATCH // NW
GROUPS = BPW // L


def _mf_body(users_hbm, items_hbm, uflat, iflat, out_hbm,
             uidx_v, iidx_v, ubufT, ibufT, out_v, sem):
    wid = lax.axis_index("s") * NC + lax.axis_index("c")
    base = wid * BPW

    pltpu.sync_copy(users_hbm.at[pl.ds(base, BPW)], uidx_v)
    pltpu.sync_copy(items_hbm.at[pl.ds(base, BPW)], iidx_v)

    def fire(g, carry):
        r0 = pl.multiple_of(g * L, L)
        ou = uidx_v[pl.ds(r0, L)]
        oi = iidx_v[pl.ds(r0, L)]
        for d in range(DIM):
            c = d * NROW
            pltpu.async_copy(uflat.at[ou + c],
                             ubufT.at[pl.ds(d * BPW + r0, L)], sem)
            pltpu.async_copy(iflat.at[oi + c],
                             ibufT.at[pl.ds(d * BPW + r0, L)], sem)
        return carry

    lax.fori_loop(0, GROUPS, fire, 0)

    pltpu.make_async_copy(uflat.at[pl.ds(0, DIM * BPW)], ubufT, sem).wait()
    pltpu.make_async_copy(iflat.at[pl.ds(0, DIM * BPW)], ibufT, sem).wait()

    def group(g, carry):
        r0 = pl.multiple_of(g * L, L)
        acc = ubufT[pl.ds(r0, L)] * ibufT[pl.ds(r0, L)]
        for d in range(1, DIM):
            acc = acc + (ubufT[pl.ds(d * BPW + r0, L)]
                         * ibufT[pl.ds(d * BPW + r0, L)])
        out_v[pl.ds(r0, L)] = 1.0 / (1.0 + jnp.exp(-acc))
        return carry

    lax.fori_loop(0, GROUPS, group, 0)

    pltpu.sync_copy(out_v, out_hbm.at[pl.ds(base, BPW)])


@jax.jit
def kernel(users, items, user_emb, item_emb):
    users = users.astype(jnp.int32)
    items = items.astype(jnp.int32)
    mesh = plsc.VectorSubcoreMesh(core_axis_name="c", subcore_axis_name="s")
    run = pl.kernel(
        _mf_body,
        out_type=jax.ShapeDtypeStruct((BATCH,), jnp.float32),
        mesh=mesh,
        compiler_params=pltpu.CompilerParams(
            needs_layout_passes=False, use_tc_tiling_on_sc=False),
        scratch_types=[
            pltpu.VMEM((BPW,), jnp.int32),
            pltpu.VMEM((BPW,), jnp.int32),
            pltpu.VMEM((DIM * BPW,), jnp.float32),
            pltpu.VMEM((DIM * BPW,), jnp.float32),
            pltpu.VMEM((BPW,), jnp.float32),
            pltpu.SemaphoreType.DMA,
        ],
    )
    return run(users, items,
               user_emb.T.reshape(DIM * NROW), item_emb.T.reshape(DIM * NROW))


# speedup vs baseline: 19.3754x; 19.3754x over previous
import jax
import jax.numpy as jnp
from jax import lax
from jax.experimental import pallas as pl
from jax.experimental.pallas import tpu as pltpu
from jax.experimental.pallas import tpu_sc as plsc

NC, NS, L = 2, 16, 16
NW = NC * NS
BATCH = 16384
DIM = 16
NROW = 1000000
BPW = BATCH // NW
GROUPS = BPW // L


def _mf_body(users_hbm, items_hbm, uemb_hbm, iemb_hbm, out_hbm,
             uidx_v, iidx_v, ublk, iblk, out_v, sem):
    wid = lax.axis_index("s") * NC + lax.axis_index("c")
    base = wid * BPW

    pltpu.sync_copy(users_hbm.at[pl.ds(base, BPW)], uidx_v)
    pltpu.sync_copy(items_hbm.at[pl.ds(base, BPW)], iidx_v)

    lanes = lax.iota(jnp.int32, L)

    def wave(g, carry):
        r0 = pl.multiple_of(g * L, L)
        ru16 = uidx_v[pl.ds(r0, L)]
        ri16 = iidx_v[pl.ds(r0, L)]
        copies = []
        for k in range(L):
            ju = pl.multiple_of((ru16[k] >> 7) * 128, 128)
            ji = pl.multiple_of((ri16[k] >> 7) * 128, 128)
            copies.append(pltpu.async_copy(
                uemb_hbm.at[:, pl.ds(ju, 128)],
                ublk.at[pl.ds(k * DIM, DIM)], sem))
            copies.append(pltpu.async_copy(
                iemb_hbm.at[:, pl.ds(ji, 128)],
                iblk.at[pl.ds(k * DIM, DIM)], sem))
        for c in copies:
            c.wait()
        ucol = ru16 & 127
        icol = ri16 & 127
        acc = None
        for d in range(DIM):
            rows = lanes * DIM + d
            u = plsc.load_gather(ublk, [rows, ucol])
            it = plsc.load_gather(iblk, [rows, icol])
            acc = u * it if acc is None else acc + u * it
        out_v[pl.ds(r0, L)] = 1.0 / (1.0 + jnp.exp(-acc))
        return carry

    lax.fori_loop(0, GROUPS, wave, 0)

    pltpu.sync_copy(out_v, out_hbm.at[pl.ds(base, BPW)])


@jax.jit
def kernel(users, items, user_emb, item_emb):
    users = users.astype(jnp.int32)
    items = items.astype(jnp.int32)
    mesh = plsc.VectorSubcoreMesh(core_axis_name="c", subcore_axis_name="s")
    run = pl.kernel(
        _mf_body,
        out_type=jax.ShapeDtypeStruct((BATCH,), jnp.float32),
        mesh=mesh,
        compiler_params=pltpu.CompilerParams(
            needs_layout_passes=False, use_tc_tiling_on_sc=True),
        scratch_types=[
            pltpu.VMEM((BPW,), jnp.int32),
            pltpu.VMEM((BPW,), jnp.int32),
            pltpu.VMEM((L * DIM, 128), jnp.float32),
            pltpu.VMEM((L * DIM, 128), jnp.float32),
            pltpu.VMEM((BPW,), jnp.float32),
            pltpu.SemaphoreType.DMA,
        ],
    )
    return run(users, items, user_emb.T, item_emb.T)


# speedup vs baseline: 19.4757x; 1.0052x over previous
import jax
import jax.numpy as jnp
from jax import lax
from jax.experimental import pallas as pl
from jax.experimental.pallas import tpu as pltpu
from jax.experimental.pallas import tpu_sc as plsc

NC, NS, L = 2, 16, 16
NW = NC * NS
BATCH = 16384
DIM = 16
HD = DIM // 2
NROW = 1000000
BPW = BATCH // NW
WAVES = BPW // L
STEPS = 2 * WAVES


def _mf_body(users_hbm, items_hbm, uemb_hbm, iemb_hbm, out_hbm,
             uidx_v, iidx_v, ublk, iblk, out_v, sems):
    wid = lax.axis_index("s") * NC + lax.axis_index("c")
    base = wid * BPW

    uemb3 = uemb_hbm.reshape(2, HD, NROW)
    iemb3 = iemb_hbm.reshape(2, HD, NROW)

    pltpu.sync_copy(users_hbm.at[pl.ds(base, BPW)], uidx_v)
    pltpu.sync_copy(items_hbm.at[pl.ds(base, BPW)], iidx_v)

    lanes = lax.iota(jnp.int32, L)

    def fire(s):
        p = s >> 5
        w = s & 31
        q = s & 1
        r0 = pl.multiple_of(w * L, L)
        ru16 = uidx_v[pl.ds(r0, L)]
        ri16 = iidx_v[pl.ds(r0, L)]
        for k in range(L):
            ju = pl.multiple_of((ru16[k] >> 7) * 128, 128)
            ji = pl.multiple_of((ri16[k] >> 7) * 128, 128)
            pltpu.async_copy(uemb3.at[p, :, pl.ds(ju, 128)],
                             ublk.at[q, pl.ds(k * HD, HD)], sems.at[q])
            pltpu.async_copy(iemb3.at[p, :, pl.ds(ji, 128)],
                             iblk.at[q, pl.ds(k * HD, HD)], sems.at[q])

    fire(jnp.int32(0))

    def step(s, carry):
        p = s >> 5
        w = s & 31
        q = s & 1

        @pl.when(s < STEPS - 1)
        def _():
            fire(s + 1)

        for k in range(L):
            pltpu.make_async_copy(uemb3.at[0, :, pl.ds(0, 128)],
                                  ublk.at[q, pl.ds(k * HD, HD)],
                                  sems.at[q]).wait()
            pltpu.make_async_copy(uemb3.at[0, :, pl.ds(0, 128)],
                                  iblk.at[q, pl.ds(k * HD, HD)],
                                  sems.at[q]).wait()

        r0 = pl.multiple_of(w * L, L)
        ucol = uidx_v[pl.ds(r0, L)] & 127
        icol = iidx_v[pl.ds(r0, L)] & 127
        ub = ublk.at[q]
        ib = iblk.at[q]
        acc = None
        for d in range(HD):
            rows = lanes * HD + d
            u = plsc.load_gather(ub, [rows, ucol])
            it = plsc.load_gather(ib, [rows, icol])
            acc = u * it if acc is None else acc + u * it

        @pl.when(p == 0)
        def _():
            out_v[pl.ds(r0, L)] = acc

        @pl.when(p == 1)
        def _():
            tot = out_v[pl.ds(r0, L)] + acc
            out_v[pl.ds(r0, L)] = 1.0 / (1.0 + jnp.exp(-tot))

        return carry

    lax.fori_loop(0, STEPS, step, 0)

    pltpu.sync_copy(out_v, out_hbm.at[pl.ds(base, BPW)])


@jax.jit
def kernel(users, items, user_emb, item_emb):
    users = users.astype(jnp.int32)
    items = items.astype(jnp.int32)
    mesh = plsc.VectorSubcoreMesh(core_axis_name="c", subcore_axis_name="s")
    run = pl.kernel(
        _mf_body,
        out_type=jax.ShapeDtypeStruct((BATCH,), jnp.float32),
        mesh=mesh,
        compiler_params=pltpu.CompilerParams(
            needs_layout_passes=False, use_tc_tiling_on_sc=True),
        scratch_types=[
            pltpu.VMEM((BPW,), jnp.int32),
            pltpu.VMEM((BPW,), jnp.int32),
            pltpu.VMEM((2, L * HD, 128), jnp.float32),
            pltpu.VMEM((2, L * HD, 128), jnp.float32),
            pltpu.VMEM((BPW,), jnp.float32),
            pltpu.SemaphoreType.DMA((2,)),
        ],
    )
    return run(users, items, user_emb.T, item_emb.T)
